# Initial kernel scaffold; baseline (speedup 1.0000x reference)
import jax
import jax.numpy as jnp
from jax.experimental import pallas as pl


def kernel(pred_p3, pred_p4, pred_p5, target_boxes, target_labels):
    raise NotImplementedError("write your pallas kernel here")


# speedup vs baseline: 4.9343x; 4.9343x over previous
import functools

import jax
import jax.numpy as jnp
from jax import lax
from jax.experimental import pallas as pl

_NC = 80
_NA = 3
_IMG = 640.0
_L_NOOBJ = 1.0
_L_OBJ = 1.0
_L_CLS = 0.5
_L_BOX = 5.0
_F_ALPHA = 0.25
_F_GAMMA = 2.0


def _bce(x, t):
    return jnp.maximum(x, 0.0) - x * t + jnp.log1p(jnp.exp(-jnp.abs(x)))


def _scale_body(pred_ref, boxes_ref, labels_ref, out_ref, *, h, w, s):
    b = pl.program_id(0)
    a = pl.program_id(1)
    hw = h * w

    boxes = boxes_ref[0]
    labels = labels_ref[0]
    nb = boxes.shape[0]

    x1 = boxes[:, 0:1]
    y1 = boxes[:, 1:2]
    x2 = boxes[:, 2:3]
    y2 = boxes[:, 3:4]

    bw = jnp.clip((x2 - x1) / _IMG, 1e-6, 1.0)
    bh = jnp.clip((y2 - y1) / _IMG, 1e-6, 1.0)
    max_side = jnp.maximum(bw, bh)
    scale_idx = jnp.clip(
        jnp.where(max_side < 0.15, 0, jnp.where(max_side < 0.45, 1, _NA - 1)),
        0, _NA - 1)
    valid = (labels >= 0) & (labels < _NC)
    sel = ((scale_idx == s) & valid).astype(jnp.float32)

    cx = jnp.clip((x1 + x2) * 0.5 / _IMG, 0.0, 1.0 - 1e-6)
    cy = jnp.clip((y1 + y2) * 0.5 / _IMG, 0.0, 1.0 - 1e-6)
    gx = cx * w
    gy = cy * h
    gi = jnp.clip(jnp.floor(gx).astype(jnp.int32), 0, w - 1)
    gj = jnp.clip(jnp.floor(gy).astype(jnp.int32), 0, h - 1)
    cell = gj * w + gi
    labels_c = jnp.clip(labels, 0, _NC - 1)

    iota_hw = lax.broadcasted_iota(jnp.int32, (nb, hw), 1)
    onehot = (iota_hw == cell).astype(jnp.float32) * sel

    match_sum = jnp.sum(onehot, axis=0, keepdims=True)
    match = (match_sum > 0.5).astype(jnp.float32)

    x_obj = pred_ref[0, 0, 4:5, :]
    t = match
    bce_o = _bce(x_obj, t)
    p = jax.nn.sigmoid(x_obj)
    p_t = p * t + (1.0 - p) * (1.0 - t)
    alpha_t = _F_ALPHA * t + (1.0 - _F_ALPHA) * (1.0 - t)
    one_m = 1.0 - p_t
    elem = bce_o * (alpha_t * one_m * one_m)
    pos_e = jnp.sum(elem * match)
    neg_e = jnp.sum(elem * (1.0 - match))
    cells = jnp.sum(match)

    label_oh = (lax.broadcasted_iota(jnp.int32, (nb, _NC), 1)
                == labels_c).astype(jnp.float32)
    t_sum = lax.dot_general(label_oh, onehot, (((0,), (0,)), ((), ())),
                            preferred_element_type=jnp.float32)
    t_cls = (t_sum > 0.5).astype(jnp.float32)
    x_cls = pred_ref[0, 0, 5:, :]
    cls_part = jnp.sum(_bce(x_cls, t_cls) * match)

    x_box = pred_ref[0, 0, 0:4, :]
    pxywh = lax.dot_general(onehot, x_box, (((1,), (1,)), ((), ())),
                            preferred_element_type=jnp.float32)
    pxy = jax.nn.sigmoid(pxywh[:, 0:2])
    pwh = jax.nn.sigmoid(pxywh[:, 2:4])
    pcx = (gi.astype(jnp.float32) + pxy[:, 0:1]) / float(w)
    pcy = (gj.astype(jnp.float32) + pxy[:, 1:2]) / float(h)
    pw = pwh[:, 0:1]
    ph = pwh[:, 1:2]
    px1 = pcx - pw * 0.5
    py1 = pcy - ph * 0.5
    px2 = pcx + pw * 0.5
    py2 = pcy + ph * 0.5
    tx1 = cx - bw * 0.5
    ty1 = cy - bh * 0.5
    tx2 = cx + bw * 0.5
    ty2 = cy + bh * 0.5
    area1 = (px2 - px1) * (py2 - py1)
    area2 = (tx2 - tx1) * (ty2 - ty1)
    iw = jnp.maximum(jnp.minimum(px2, tx2) - jnp.maximum(px1, tx1), 0.0)
    ih = jnp.maximum(jnp.minimum(py2, ty2) - jnp.maximum(py1, ty1), 0.0)
    inter = iw * ih
    union = area1 + area2 - inter
    iou = inter / union
    cw = jnp.maximum(px2, tx2) - jnp.minimum(px1, tx1)
    chh = jnp.maximum(py2, ty2) - jnp.minimum(py1, ty1)
    areac = jnp.maximum(cw, 0.0) * jnp.maximum(chh, 0.0)
    giou = iou - (areac - union) / areac
    box_part = jnp.sum((1.0 - giou) * sel)
    selsum = jnp.sum(sel)

    lane = lax.broadcasted_iota(jnp.int32, (1, 128), 1)
    vals = (pos_e, neg_e, cells, cls_part, box_part, selsum)
    row = jnp.zeros((1, 128), jnp.float32)
    for k, v in enumerate(vals):
        row = row + jnp.where(lane == k, v, 0.0)

    @pl.when((b == 0) & (a == 0))
    def _():
        out_ref[...] = jnp.zeros_like(out_ref)

    out_ref[...] += row


def _run_scale(pred, boxes, labels3, s, h, w):
    B = pred.shape[0]
    hw = h * w
    pred4 = pred.reshape(B, _NA, 5 + _NC, hw)
    return pl.pallas_call(
        functools.partial(_scale_body, h=h, w=w, s=s),
        grid=(B, _NA),
        in_specs=[
            pl.BlockSpec((1, 1, 5 + _NC, hw), lambda b, a: (b, a, 0, 0)),
            pl.BlockSpec((1, 32, 4), lambda b, a: (b, 0, 0)),
            pl.BlockSpec((1, 32, 1), lambda b, a: (b, 0, 0)),
        ],
        out_specs=pl.BlockSpec((1, 128), lambda b, a: (0, 0)),
        out_shape=jax.ShapeDtypeStruct((1, 128), jnp.float32),
    )(pred4, boxes, labels3)


def kernel(pred_p3, pred_p4, pred_p5, target_boxes, target_labels):
    labels3 = target_labels.reshape(target_labels.shape[0], -1, 1)
    o3 = _run_scale(pred_p3, target_boxes, labels3, 0, 80, 80)
    o4 = _run_scale(pred_p4, target_boxes, labels3, 1, 40, 40)
    o5 = _run_scale(pred_p5, target_boxes, labels3, 2, 20, 20)

    obj = jnp.float32(0.0)
    cls = jnp.float32(0.0)
    box = jnp.float32(0.0)
    totpos = jnp.float32(0.0)
    for o in (o3, o4, o5):
        r = o[0]
        denom = jnp.maximum(r[2], 1.0)
        obj = obj + _L_OBJ * r[0] / denom + _L_NOOBJ * r[1] / denom
        cls = cls + r[3] / jnp.maximum(r[2] * _NC, 1.0)
        box = box + r[4]
        totpos = totpos + r[5]
    box = box / jnp.maximum(totpos, 1.0)
    return obj + _L_CLS * cls + _L_BOX * box
